# baseline (speedup 1.0000x reference)
import functools

import jax
import jax.numpy as jnp
from jax import lax
from jax.experimental import pallas as pl
from jax.experimental.pallas import tpu as pltpu
from jax.experimental.pallas import tpu_sc as plsc

_B, _S = 4096, 200
_D = 64
_N = _B * _S
_NC, _NS = 2, 16
_NW = _NC * _NS
_PER_W = _N // _NW
_CHUNK = 128
_NCHUNK = _PER_W // _CHUNK

_mesh = plsc.VectorSubcoreMesh(core_axis_name="c", subcore_axis_name="s")


@functools.partial(
    pl.kernel,
    out_type=jax.ShapeDtypeStruct((_N, _D), jnp.float32),
    mesh=_mesh,
    scratch_types=[
        pltpu.VMEM((_NCHUNK, _CHUNK), jnp.int32),
        pltpu.VMEM((_CHUNK, _D), jnp.float32),
        pltpu.SemaphoreType.DMA,
    ],
    compiler_params=pltpu.CompilerParams(use_tc_tiling_on_sc=False),
)
def _gather_kernel(idx_hbm, table_hbm, out_hbm, idx_v, rows_v, sem):
    wid = lax.axis_index("s") * _NC + lax.axis_index("c")
    base = wid * _PER_W
    pltpu.sync_copy(idx_hbm.at[wid], idx_v)

    def body(i, _):
        pltpu.async_copy(table_hbm.at[idx_v.at[i]], rows_v, sem).wait()
        pltpu.sync_copy(rows_v, out_hbm.at[pl.ds(base + i * _CHUNK, _CHUNK)])
        return ()

    lax.fori_loop(0, _NCHUNK, body, ())


def kernel(token_ids, weights):
    idx = token_ids.reshape(_NW, _NCHUNK, _CHUNK).astype(jnp.int32)
    out = _gather_kernel(idx, weights)
    return out.reshape(_B, _S, _D)


# speedup vs baseline: 1.1138x; 1.1138x over previous
import functools

import jax
import jax.numpy as jnp
from jax import lax
from jax.experimental import pallas as pl
from jax.experimental.pallas import tpu as pltpu
from jax.experimental.pallas import tpu_sc as plsc

_B, _S = 4096, 200
_D = 64
_N = _B * _S
_NC, _NS = 2, 16
_NW = _NC * _NS
_PER_W = _N // _NW
_CHUNK = 128
_NCHUNK = _PER_W // _CHUNK
_NBUF = 4

_mesh = plsc.VectorSubcoreMesh(core_axis_name="c", subcore_axis_name="s")


@functools.partial(
    pl.kernel,
    out_type=jax.ShapeDtypeStruct((_N, _D), jnp.float32),
    mesh=_mesh,
    scratch_types=[
        pltpu.VMEM((_NCHUNK, _CHUNK), jnp.int32),
        pltpu.VMEM((_NBUF, _CHUNK, _D), jnp.float32),
        pltpu.SemaphoreType.DMA,
        pltpu.SemaphoreType.DMA,
    ],
    compiler_params=pltpu.CompilerParams(use_tc_tiling_on_sc=False),
)
def _gather_kernel(idx_hbm, table_hbm, out_hbm, idx_v, rows_v, sem_g, sem_s):
    wid = lax.axis_index("s") * _NC + lax.axis_index("c")
    base = wid * _PER_W
    pltpu.sync_copy(idx_hbm.at[wid], idx_v)

    def gather_start(i, slot):
        pltpu.async_copy(table_hbm.at[idx_v.at[i]], rows_v.at[slot], sem_g)

    def gather_wait(slot):
        pltpu.make_async_copy(
            out_hbm.at[pl.ds(base, _CHUNK)], rows_v.at[slot], sem_g
        ).wait()

    def store_start(i, slot):
        pltpu.async_copy(
            rows_v.at[slot], out_hbm.at[pl.ds(base + i * _CHUNK, _CHUNK)], sem_s
        )

    def store_wait(slot):
        pltpu.make_async_copy(
            out_hbm.at[pl.ds(base, _CHUNK)], rows_v.at[slot], sem_s
        ).wait()

    gather_start(0, 0)
    gather_start(1, 1)
    gather_start(2, 2)
    gather_wait(0)
    store_start(0, 0)
    gather_start(3, 3)

    def body(t, _):
        for p in range(_NBUF):
            i = 1 + t * _NBUF + p
            slot = (1 + p) % _NBUF
            gather_wait(slot)
            store_start(i, slot)
            store_wait((p) % _NBUF)
            gather_start(i + 3, p % _NBUF)
        return ()

    lax.fori_loop(0, (_NCHUNK - _NBUF) // _NBUF, body, ())

    for i in range(_NCHUNK - 3, _NCHUNK):
        slot = i % _NBUF
        gather_wait(slot)
        store_start(i, slot)
        store_wait((i - 1) % _NBUF)
    store_wait((_NCHUNK - 1) % _NBUF)


def kernel(token_ids, weights):
    idx = token_ids.reshape(_NW, _NCHUNK, _CHUNK).astype(jnp.int32)
    out = _gather_kernel(idx, weights)
    return out.reshape(_B, _S, _D)


# speedup vs baseline: 1.1147x; 1.0008x over previous
import functools

import jax
import jax.numpy as jnp
from jax import lax
from jax.experimental import pallas as pl
from jax.experimental.pallas import tpu as pltpu
from jax.experimental.pallas import tpu_sc as plsc

_B, _S = 4096, 200
_D = 64
_N = _B * _S
_NC, _NS = 2, 16
_NW = _NC * _NS
_PER_W = _N // _NW
_CHUNK = 256
_NCHUNK = _PER_W // _CHUNK
_NBUF = 4

_mesh = plsc.VectorSubcoreMesh(core_axis_name="c", subcore_axis_name="s")


@functools.partial(
    pl.kernel,
    out_type=jax.ShapeDtypeStruct((_N, _D), jnp.float32),
    mesh=_mesh,
    scratch_types=[
        pltpu.VMEM((_NCHUNK, _CHUNK), jnp.int32),
        pltpu.VMEM((_NBUF, _CHUNK, _D), jnp.float32),
        pltpu.SemaphoreType.DMA,
        pltpu.SemaphoreType.DMA,
    ],
    compiler_params=pltpu.CompilerParams(use_tc_tiling_on_sc=False),
)
def _gather_kernel(idx_hbm, table_hbm, out_hbm, idx_v, rows_v, sem_g, sem_s):
    wid = lax.axis_index("s") * _NC + lax.axis_index("c")
    base = wid * _PER_W
    pltpu.sync_copy(idx_hbm.at[wid], idx_v)

    def gather_start(i, slot):
        pltpu.async_copy(table_hbm.at[idx_v.at[i]], rows_v.at[slot], sem_g)

    def gather_wait(slot):
        pltpu.make_async_copy(
            out_hbm.at[pl.ds(base, _CHUNK)], rows_v.at[slot], sem_g
        ).wait()

    def store_start(i, slot):
        pltpu.async_copy(
            rows_v.at[slot], out_hbm.at[pl.ds(base + i * _CHUNK, _CHUNK)], sem_s
        )

    def store_wait(slot):
        pltpu.make_async_copy(
            out_hbm.at[pl.ds(base, _CHUNK)], rows_v.at[slot], sem_s
        ).wait()

    gather_start(0, 0)
    gather_start(1, 1)
    gather_start(2, 2)
    gather_wait(0)
    store_start(0, 0)
    gather_start(3, 3)

    def body(t, _):
        for p in range(_NBUF):
            i = 1 + t * _NBUF + p
            slot = (1 + p) % _NBUF
            gather_wait(slot)
            store_start(i, slot)
            store_wait((p) % _NBUF)
            gather_start(i + 3, p % _NBUF)
        return ()

    lax.fori_loop(0, (_NCHUNK - _NBUF) // _NBUF, body, ())

    for i in range(_NCHUNK - 3, _NCHUNK):
        slot = i % _NBUF
        gather_wait(slot)
        store_start(i, slot)
        store_wait((i - 1) % _NBUF)
    store_wait((_NCHUNK - 1) % _NBUF)


def kernel(token_ids, weights):
    idx = token_ids.reshape(_NW, _NCHUNK, _CHUNK).astype(jnp.int32)
    out = _gather_kernel(idx, weights)
    return out.reshape(_B, _S, _D)
